# baseline (speedup 1.0000x reference)
import math

import jax
import jax.numpy as jnp
from jax.experimental import pallas as pl
from jax.experimental.pallas import tpu as pltpu

_PI = math.pi
_TWO_PI = 2.0 * math.pi
_HALF_PI = 0.5 * math.pi
_INV_TWO_PI = 1.0 / _TWO_PI

_ATAN_C = (
    0.9999966346599344,
    -0.3331830275252533,
    0.19813212106599729,
    -0.1324751723201036,
    0.07981110084304613,
    -0.033725845571015184,
    0.006842593618516107,
)


def _wrap(a):
    return a - _TWO_PI * jnp.floor((a + _PI) * _INV_TWO_PI)


def _atan2(y, x):
    ax = jnp.abs(x)
    ay = jnp.abs(y)
    hi = jnp.maximum(ax, ay)
    lo = jnp.minimum(ax, ay)
    a = lo / jnp.where(hi == 0.0, 1.0, hi)
    s = a * a
    p = jnp.float32(_ATAN_C[6])
    for c in (_ATAN_C[5], _ATAN_C[4], _ATAN_C[3], _ATAN_C[2], _ATAN_C[1],
              _ATAN_C[0]):
        p = p * s + jnp.float32(c)
    r = a * p
    r = jnp.where(ay > ax, _HALF_PI - r, r)
    r = jnp.where(x < 0.0, _PI - r, r)
    return jnp.where(y < 0.0, -r, r)


def _geom_kernel(plx_ref, ply_ref, opl_ref, vpl_ref, ptxy_ref, opt_ref,
                 r_pl2pl_ref, r_pt2pl_ref, mask_ref):
    r = pl.program_id(1)
    R = r_pl2pl_ref.shape[2]
    base = r * R

    xj = plx_ref[0, 0, :]
    yj = ply_ref[0, 0, :]
    oj = opl_ref[0, 0, :]
    vj = vpl_ref[0, 0, :]

    xi = plx_ref[0, 0, pl.ds(base, R)]
    yi = ply_ref[0, 0, pl.ds(base, R)]
    oi = opl_ref[0, 0, pl.ds(base, R)]
    vi = vpl_ref[0, 0, pl.ds(base, R)]

    oi_col = oi[:, None]

    dx = xj[None, :] - xi[:, None]
    dy = yj[None, :] - yi[:, None]
    r_pl2pl_ref[0, 0, :, :] = jnp.sqrt(dx * dx + dy * dy)
    r_pl2pl_ref[0, 1, :, :] = _wrap(_atan2(dy, dx) - oi_col)
    r_pl2pl_ref[0, 2, :, :] = _wrap(oi_col - oj[None, :])

    n = xj.shape[0]
    row = jax.lax.broadcasted_iota(jnp.int32, (R, n), 0) + base
    col = jax.lax.broadcasted_iota(jnp.int32, (R, n), 1)
    mask_ref[0, :, :] = (vi[:, None] > 0.0) & (vj[None, :] > 0.0) & (row != col)

    ptxy = ptxy_ref[0, :, :]
    npt = ptxy.shape[1] // 2
    sel_row = jax.lax.broadcasted_iota(jnp.int32, (2 * npt, npt), 0)
    sel_col = jax.lax.broadcasted_iota(jnp.int32, (2 * npt, npt), 1)
    p_even = (sel_row == 2 * sel_col).astype(jnp.float32)
    p_odd = (sel_row == 2 * sel_col + 1).astype(jnp.float32)
    px = jax.lax.dot(ptxy, p_even, precision=jax.lax.Precision.HIGHEST)
    py = jax.lax.dot(ptxy, p_odd, precision=jax.lax.Precision.HIGHEST)
    dxp = px * 0.1 - xi[:, None]
    dyp = py * 0.1 - yi[:, None]
    r_pt2pl_ref[0, 0, :, :] = jnp.sqrt(dxp * dxp + dyp * dyp)
    r_pt2pl_ref[0, 1, :, :] = _wrap(_atan2(dyp, dxp) - oi_col)
    r_pt2pl_ref[0, 2, :, :] = _wrap(opt_ref[0, :, :] - oi_col)


def kernel(pos_pt, orient_pt, pos_pl, orient_pl, valid_pl):
    B, PL, PT, _ = pos_pt.shape
    R = 128

    ptxy = pos_pt.reshape(B, PL, 2 * PT)
    plx = (pos_pl[..., 0] * 0.1).reshape(B, 1, PL)
    ply = (pos_pl[..., 1] * 0.1).reshape(B, 1, PL)
    opl = orient_pl.reshape(B, 1, PL)
    vpl = valid_pl.astype(jnp.float32).reshape(B, 1, PL)

    pl_row_spec = pl.BlockSpec((1, 1, PL), lambda b, r: (b, 0, 0))
    ptxy_spec = pl.BlockSpec((1, R, 2 * PT), lambda b, r: (b, r, 0))
    opt_spec = pl.BlockSpec((1, R, PT), lambda b, r: (b, r, 0))

    r_pl2pl, r_pt2pl, mask = pl.pallas_call(
        _geom_kernel,
        grid=(B, PL // R),
        in_specs=[pl_row_spec, pl_row_spec, pl_row_spec, pl_row_spec,
                  ptxy_spec, opt_spec],
        out_specs=[
            pl.BlockSpec((1, 3, R, PL), lambda b, r: (b, 0, r, 0)),
            pl.BlockSpec((1, 3, R, PT), lambda b, r: (b, 0, r, 0)),
            pl.BlockSpec((1, R, PL), lambda b, r: (b, r, 0)),
        ],
        out_shape=(
            jax.ShapeDtypeStruct((B, 3, PL, PL), jnp.float32),
            jax.ShapeDtypeStruct((B, 3, PL, PT), jnp.float32),
            jax.ShapeDtypeStruct((B, PL, PL), jnp.bool_),
        ),
        compiler_params=pltpu.CompilerParams(
            dimension_semantics=("parallel", "parallel"),
        ),
    )(plx, ply, opl, vpl, ptxy, orient_pt)

    return (r_pl2pl, r_pt2pl, mask)


# speedup vs baseline: 1.0973x; 1.0973x over previous
import math

import jax
import jax.numpy as jnp
from jax.experimental import pallas as pl
from jax.experimental.pallas import tpu as pltpu

_PI = math.pi
_TWO_PI = 2.0 * math.pi
_HALF_PI = 0.5 * math.pi
_INV_TWO_PI = 1.0 / _TWO_PI

_ATAN_C = (
    0.9999966346599344,
    -0.3331830275252533,
    0.19813212106599729,
    -0.1324751723201036,
    0.07981110084304613,
    -0.033725845571015184,
    0.006842593618516107,
)


def _wrap(a):
    return a - _TWO_PI * jnp.floor((a + _PI) * _INV_TWO_PI)


def _atan2(y, x):
    ax = jnp.abs(x)
    ay = jnp.abs(y)
    hi = jnp.maximum(ax, ay)
    lo = jnp.minimum(ax, ay)
    a = lo / jnp.where(hi == 0.0, 1.0, hi)
    s = a * a
    p = jnp.float32(_ATAN_C[6])
    for c in (_ATAN_C[5], _ATAN_C[4], _ATAN_C[3], _ATAN_C[2], _ATAN_C[1],
              _ATAN_C[0]):
        p = p * s + jnp.float32(c)
    r = a * p
    r = jnp.where(ay > ax, _HALF_PI - r, r)
    r = jnp.where(x < 0.0, _PI - r, r)
    return jnp.where(y < 0.0, -r, r)


def _geom_kernel(plx_ref, ply_ref, opl_ref, vpl_ref, ptxy_ref, opt_ref,
                 r_pl2pl_ref, r_pt2pl_ref, mask_ref):
    r = pl.program_id(1)
    R = r_pl2pl_ref.shape[2]
    base = r * R

    xj = plx_ref[0, 0, :]
    yj = ply_ref[0, 0, :]
    oj = opl_ref[0, 0, :]
    vj = vpl_ref[0, 0, :]

    xi = plx_ref[0, 0, pl.ds(base, R)]
    yi = ply_ref[0, 0, pl.ds(base, R)]
    oi = opl_ref[0, 0, pl.ds(base, R)]
    vi = vpl_ref[0, 0, pl.ds(base, R)]

    oi_col = oi[:, None]

    dx = xj[None, :] - xi[:, None]
    dy = yj[None, :] - yi[:, None]
    r_pl2pl_ref[0, 0, :, :] = jnp.sqrt(dx * dx + dy * dy)
    r_pl2pl_ref[0, 1, :, :] = _wrap(_atan2(dy, dx) - oi_col)
    r_pl2pl_ref[0, 2, :, :] = _wrap(oi_col - oj[None, :])

    n = xj.shape[0]
    row = jax.lax.broadcasted_iota(jnp.int32, (R, n), 0) + base
    col = jax.lax.broadcasted_iota(jnp.int32, (R, n), 1)
    mask_ref[0, :, :] = (vi[:, None] > 0.0) & (vj[None, :] > 0.0) & (row != col)

    ptxy = ptxy_ref[0, :, :]
    npt = ptxy.shape[1] // 2
    sel_row = jax.lax.broadcasted_iota(jnp.int32, (2 * npt, npt), 0)
    sel_col = jax.lax.broadcasted_iota(jnp.int32, (2 * npt, npt), 1)
    p_even = (sel_row == 2 * sel_col).astype(jnp.float32)
    p_odd = (sel_row == 2 * sel_col + 1).astype(jnp.float32)
    px = jax.lax.dot(ptxy, p_even, precision=jax.lax.Precision.HIGHEST)
    py = jax.lax.dot(ptxy, p_odd, precision=jax.lax.Precision.HIGHEST)
    dxp = px * 0.1 - xi[:, None]
    dyp = py * 0.1 - yi[:, None]
    r_pt2pl_ref[0, 0, :, :] = jnp.sqrt(dxp * dxp + dyp * dyp)
    r_pt2pl_ref[0, 1, :, :] = _wrap(_atan2(dyp, dxp) - oi_col)
    r_pt2pl_ref[0, 2, :, :] = _wrap(opt_ref[0, :, :] - oi_col)


def kernel(pos_pt, orient_pt, pos_pl, orient_pl, valid_pl):
    B, PL, PT, _ = pos_pt.shape
    R = 256

    ptxy = pos_pt.reshape(B, PL, 2 * PT)
    plx = (pos_pl[..., 0] * 0.1).reshape(B, 1, PL)
    ply = (pos_pl[..., 1] * 0.1).reshape(B, 1, PL)
    opl = orient_pl.reshape(B, 1, PL)
    vpl = valid_pl.astype(jnp.float32).reshape(B, 1, PL)

    pl_row_spec = pl.BlockSpec((1, 1, PL), lambda b, r: (b, 0, 0))
    ptxy_spec = pl.BlockSpec((1, R, 2 * PT), lambda b, r: (b, r, 0))
    opt_spec = pl.BlockSpec((1, R, PT), lambda b, r: (b, r, 0))

    r_pl2pl, r_pt2pl, mask = pl.pallas_call(
        _geom_kernel,
        grid=(B, PL // R),
        in_specs=[pl_row_spec, pl_row_spec, pl_row_spec, pl_row_spec,
                  ptxy_spec, opt_spec],
        out_specs=[
            pl.BlockSpec((1, 3, R, PL), lambda b, r: (b, 0, r, 0)),
            pl.BlockSpec((1, 3, R, PT), lambda b, r: (b, 0, r, 0)),
            pl.BlockSpec((1, R, PL), lambda b, r: (b, r, 0)),
        ],
        out_shape=(
            jax.ShapeDtypeStruct((B, 3, PL, PL), jnp.float32),
            jax.ShapeDtypeStruct((B, 3, PL, PT), jnp.float32),
            jax.ShapeDtypeStruct((B, PL, PL), jnp.bool_),
        ),
        compiler_params=pltpu.CompilerParams(
            dimension_semantics=("parallel", "parallel"),
        ),
    )(plx, ply, opl, vpl, ptxy, orient_pt)

    return (r_pl2pl, r_pt2pl, mask)


# speedup vs baseline: 1.1320x; 1.0316x over previous
import math

import jax
import jax.numpy as jnp
from jax.experimental import pallas as pl
from jax.experimental.pallas import tpu as pltpu

_PI = math.pi
_TWO_PI = 2.0 * math.pi
_HALF_PI = 0.5 * math.pi
_INV_TWO_PI = 1.0 / _TWO_PI

_ATAN_C = (
    0.9999966346599344,
    -0.3331830275252533,
    0.19813212106599729,
    -0.1324751723201036,
    0.07981110084304613,
    -0.033725845571015184,
    0.006842593618516107,
)


def _wrap(a):
    return a - _TWO_PI * jnp.floor((a + _PI) * _INV_TWO_PI)


def _atan2(y, x):
    ax = jnp.abs(x)
    ay = jnp.abs(y)
    hi = jnp.maximum(ax, ay)
    lo = jnp.minimum(ax, ay)
    a = lo / jnp.where(hi == 0.0, 1.0, hi)
    s = a * a
    p = jnp.float32(_ATAN_C[6])
    for c in (_ATAN_C[5], _ATAN_C[4], _ATAN_C[3], _ATAN_C[2], _ATAN_C[1],
              _ATAN_C[0]):
        p = p * s + jnp.float32(c)
    r = a * p
    r = jnp.where(ay > ax, _HALF_PI - r, r)
    r = jnp.where(x < 0.0, _PI - r, r)
    return jnp.where(y < 0.0, -r, r)


def _geom_kernel(plx_ref, ply_ref, opl_ref, vpl_ref, ptxy_ref, opt_ref,
                 r_pl2pl_ref, r_pt2pl_ref, mask_ref):
    r = pl.program_id(1)
    R = r_pl2pl_ref.shape[2]
    base = r * R

    xj = plx_ref[0, 0, :]
    yj = ply_ref[0, 0, :]
    oj = opl_ref[0, 0, :]
    vj = vpl_ref[0, 0, :]

    xi = plx_ref[0, 0, pl.ds(base, R)]
    yi = ply_ref[0, 0, pl.ds(base, R)]
    oi = opl_ref[0, 0, pl.ds(base, R)]
    vi = vpl_ref[0, 0, pl.ds(base, R)]

    oi_col = oi[:, None]

    dx = xj[None, :] - xi[:, None]
    dy = yj[None, :] - yi[:, None]
    r_pl2pl_ref[0, 0, :, :] = jnp.sqrt(dx * dx + dy * dy)
    r_pl2pl_ref[0, 1, :, :] = _wrap(_atan2(dy, dx) - oi_col)
    r_pl2pl_ref[0, 2, :, :] = _wrap(oi_col - oj[None, :])

    n = xj.shape[0]
    row = jax.lax.broadcasted_iota(jnp.int32, (R, n), 0) + base
    col = jax.lax.broadcasted_iota(jnp.int32, (R, n), 1)
    mask_ref[0, :, :] = (vi[:, None] > 0.0) & (vj[None, :] > 0.0) & (row != col)

    ptxy = ptxy_ref[0, :, :]
    npt = ptxy.shape[1] // 2
    sel_row = jax.lax.broadcasted_iota(jnp.int32, (2 * npt, npt), 0)
    sel_col = jax.lax.broadcasted_iota(jnp.int32, (2 * npt, npt), 1)
    p_even = (sel_row == 2 * sel_col).astype(jnp.float32)
    p_odd = (sel_row == 2 * sel_col + 1).astype(jnp.float32)
    px = jax.lax.dot(ptxy, p_even, precision=jax.lax.Precision.HIGHEST)
    py = jax.lax.dot(ptxy, p_odd, precision=jax.lax.Precision.HIGHEST)
    dxp = px * 0.1 - xi[:, None]
    dyp = py * 0.1 - yi[:, None]
    r_pt2pl_ref[0, 0, :, :] = jnp.sqrt(dxp * dxp + dyp * dyp)
    r_pt2pl_ref[0, 1, :, :] = _wrap(_atan2(dyp, dxp) - oi_col)
    r_pt2pl_ref[0, 2, :, :] = _wrap(opt_ref[0, :, :] - oi_col)


def kernel(pos_pt, orient_pt, pos_pl, orient_pl, valid_pl):
    B, PL, PT, _ = pos_pt.shape
    R = 512

    ptxy = pos_pt.reshape(B, PL, 2 * PT)
    plx = (pos_pl[..., 0] * 0.1).reshape(B, 1, PL)
    ply = (pos_pl[..., 1] * 0.1).reshape(B, 1, PL)
    opl = orient_pl.reshape(B, 1, PL)
    vpl = valid_pl.astype(jnp.float32).reshape(B, 1, PL)

    pl_row_spec = pl.BlockSpec((1, 1, PL), lambda b, r: (b, 0, 0))
    ptxy_spec = pl.BlockSpec((1, R, 2 * PT), lambda b, r: (b, r, 0))
    opt_spec = pl.BlockSpec((1, R, PT), lambda b, r: (b, r, 0))

    r_pl2pl, r_pt2pl, mask = pl.pallas_call(
        _geom_kernel,
        grid=(B, PL // R),
        in_specs=[pl_row_spec, pl_row_spec, pl_row_spec, pl_row_spec,
                  ptxy_spec, opt_spec],
        out_specs=[
            pl.BlockSpec((1, 3, R, PL), lambda b, r: (b, 0, r, 0)),
            pl.BlockSpec((1, 3, R, PT), lambda b, r: (b, 0, r, 0)),
            pl.BlockSpec((1, R, PL), lambda b, r: (b, r, 0)),
        ],
        out_shape=(
            jax.ShapeDtypeStruct((B, 3, PL, PL), jnp.float32),
            jax.ShapeDtypeStruct((B, 3, PL, PT), jnp.float32),
            jax.ShapeDtypeStruct((B, PL, PL), jnp.bool_),
        ),
        compiler_params=pltpu.CompilerParams(
            dimension_semantics=("parallel", "parallel"),
        ),
    )(plx, ply, opl, vpl, ptxy, orient_pt)

    return (r_pl2pl, r_pt2pl, mask)


# speedup vs baseline: 1.1766x; 1.0394x over previous
import math

import jax
import jax.numpy as jnp
from jax.experimental import pallas as pl
from jax.experimental.pallas import tpu as pltpu

_PI = math.pi
_TWO_PI = 2.0 * math.pi
_HALF_PI = 0.5 * math.pi
_INV_TWO_PI = 1.0 / _TWO_PI

_ATAN_C = (
    0.9999966346599344,
    -0.3331830275252533,
    0.19813212106599729,
    -0.1324751723201036,
    0.07981110084304613,
    -0.033725845571015184,
    0.006842593618516107,
)


def _wrap(a):
    return a - _TWO_PI * jnp.floor((a + _PI) * _INV_TWO_PI)


def _atan2(y, x):
    ax = jnp.abs(x)
    ay = jnp.abs(y)
    hi = jnp.maximum(ax, ay)
    lo = jnp.minimum(ax, ay)
    a = lo / jnp.where(hi == 0.0, 1.0, hi)
    s = a * a
    p = jnp.float32(_ATAN_C[6])
    for c in (_ATAN_C[5], _ATAN_C[4], _ATAN_C[3], _ATAN_C[2], _ATAN_C[1],
              _ATAN_C[0]):
        p = p * s + jnp.float32(c)
    r = a * p
    r = jnp.where(ay > ax, _HALF_PI - r, r)
    r = jnp.where(x < 0.0, _PI - r, r)
    return jnp.where(y < 0.0, -r, r)


def _geom_kernel(plx_ref, ply_ref, opl_ref, vpl_ref, ptx_ref, pty_ref, opt_ref,
                 r_pl2pl_ref, r_pt2pl_ref, mask_ref):
    r = pl.program_id(1)
    R = r_pl2pl_ref.shape[2]
    base = r * R

    xj = plx_ref[0, 0, :]
    yj = ply_ref[0, 0, :]
    oj = opl_ref[0, 0, :]
    vj = vpl_ref[0, 0, :]

    xi = plx_ref[0, 0, pl.ds(base, R)]
    yi = ply_ref[0, 0, pl.ds(base, R)]
    oi = opl_ref[0, 0, pl.ds(base, R)]
    vi = vpl_ref[0, 0, pl.ds(base, R)]

    oi_col = oi[:, None]

    dx = xj[None, :] - xi[:, None]
    dy = yj[None, :] - yi[:, None]
    r_pl2pl_ref[0, 0, :, :] = jnp.sqrt(dx * dx + dy * dy)
    r_pl2pl_ref[0, 1, :, :] = _wrap(_atan2(dy, dx) - oi_col)
    r_pl2pl_ref[0, 2, :, :] = _wrap(oi_col - oj[None, :])

    n = xj.shape[0]
    row = jax.lax.broadcasted_iota(jnp.int32, (R, n), 0) + base
    col = jax.lax.broadcasted_iota(jnp.int32, (R, n), 1)
    mask_ref[0, :, :] = (vi[:, None] > 0.0) & (vj[None, :] > 0.0) & (row != col)

    dxp = ptx_ref[0, :, :] - xi[:, None]
    dyp = pty_ref[0, :, :] - yi[:, None]
    r_pt2pl_ref[0, 0, :, :] = jnp.sqrt(dxp * dxp + dyp * dyp)
    r_pt2pl_ref[0, 1, :, :] = _wrap(_atan2(dyp, dxp) - oi_col)
    r_pt2pl_ref[0, 2, :, :] = _wrap(opt_ref[0, :, :] - oi_col)


def kernel(pos_pt, orient_pt, pos_pl, orient_pl, valid_pl):
    B, PL, PT, _ = pos_pt.shape
    R = 512

    ptx = pos_pt[..., 0] * 0.1
    pty = pos_pt[..., 1] * 0.1
    plx = (pos_pl[..., 0] * 0.1).reshape(B, 1, PL)
    ply = (pos_pl[..., 1] * 0.1).reshape(B, 1, PL)
    opl = orient_pl.reshape(B, 1, PL)
    vpl = valid_pl.astype(jnp.float32).reshape(B, 1, PL)

    pl_row_spec = pl.BlockSpec((1, 1, PL), lambda b, r: (b, 0, 0))
    pt_spec = pl.BlockSpec((1, R, PT), lambda b, r: (b, r, 0))

    r_pl2pl, r_pt2pl, mask = pl.pallas_call(
        _geom_kernel,
        grid=(B, PL // R),
        in_specs=[pl_row_spec, pl_row_spec, pl_row_spec, pl_row_spec,
                  pt_spec, pt_spec, pt_spec],
        out_specs=[
            pl.BlockSpec((1, 3, R, PL), lambda b, r: (b, 0, r, 0)),
            pl.BlockSpec((1, 3, R, PT), lambda b, r: (b, 0, r, 0)),
            pl.BlockSpec((1, R, PL), lambda b, r: (b, r, 0)),
        ],
        out_shape=(
            jax.ShapeDtypeStruct((B, 3, PL, PL), jnp.float32),
            jax.ShapeDtypeStruct((B, 3, PL, PT), jnp.float32),
            jax.ShapeDtypeStruct((B, PL, PL), jnp.bool_),
        ),
        compiler_params=pltpu.CompilerParams(
            dimension_semantics=("parallel", "parallel"),
        ),
    )(plx, ply, opl, vpl, ptx, pty, orient_pt)

    return (r_pl2pl, r_pt2pl, mask)


# speedup vs baseline: 1.2356x; 1.0502x over previous
import math

import jax
import jax.numpy as jnp
from jax.experimental import pallas as pl
from jax.experimental.pallas import tpu as pltpu

_PI = math.pi
_TWO_PI = 2.0 * math.pi
_HALF_PI = 0.5 * math.pi
_INV_TWO_PI = 1.0 / _TWO_PI

_ATAN_C = (
    0.9999966346599344,
    -0.3331830275252533,
    0.19813212106599729,
    -0.1324751723201036,
    0.07981110084304613,
    -0.033725845571015184,
    0.006842593618516107,
)


def _wrap(a):
    return a - _TWO_PI * jnp.floor((a + _PI) * _INV_TWO_PI)


def _atan2(y, x):
    ax = jnp.abs(x)
    ay = jnp.abs(y)
    hi = jnp.maximum(ax, ay)
    lo = jnp.minimum(ax, ay)
    a = lo / jnp.where(hi == 0.0, 1.0, hi)
    s = a * a
    p = jnp.float32(_ATAN_C[6])
    for c in (_ATAN_C[5], _ATAN_C[4], _ATAN_C[3], _ATAN_C[2], _ATAN_C[1],
              _ATAN_C[0]):
        p = p * s + jnp.float32(c)
    r = a * p
    r = jnp.where(ay > ax, _HALF_PI - r, r)
    r = jnp.where(x < 0.0, _PI - r, r)
    return jnp.where(y < 0.0, -r, r)


def _geom_kernel(prm_ref, ptxy_ref, opt_ref,
                 r_pl2pl_ref, r_pt2pl_ref, mask_ref):
    r = pl.program_id(1)
    R = r_pl2pl_ref.shape[2]
    base = r * R

    xj = prm_ref[0, 0, :]
    yj = prm_ref[0, 1, :]
    oj = prm_ref[0, 2, :]
    vj = prm_ref[0, 3, :]

    xi = prm_ref[0, 0, pl.ds(base, R)]
    yi = prm_ref[0, 1, pl.ds(base, R)]
    oi = prm_ref[0, 2, pl.ds(base, R)]
    vi = prm_ref[0, 3, pl.ds(base, R)]

    oi_col = oi[:, None]

    dx = xj[None, :] - xi[:, None]
    dy = yj[None, :] - yi[:, None]
    r_pl2pl_ref[0, 0, :, :] = jnp.sqrt(dx * dx + dy * dy)
    r_pl2pl_ref[0, 1, :, :] = _wrap(_atan2(dy, dx) - oi_col)
    r_pl2pl_ref[0, 2, :, :] = _wrap(oi_col - oj[None, :])

    n = xj.shape[0]
    row = jax.lax.broadcasted_iota(jnp.int32, (R, n), 0) + base
    col = jax.lax.broadcasted_iota(jnp.int32, (R, n), 1)
    mask_ref[0, :, :] = (vi[:, None] > 0.0) & (vj[None, :] > 0.0) & (row != col)

    npt = opt_ref.shape[2]
    dxp = ptxy_ref[0, :, :npt] - xi[:, None]
    dyp = ptxy_ref[0, :, npt:] - yi[:, None]
    r_pt2pl_ref[0, 0, :, :] = jnp.sqrt(dxp * dxp + dyp * dyp)
    r_pt2pl_ref[0, 1, :, :] = _wrap(_atan2(dyp, dxp) - oi_col)
    r_pt2pl_ref[0, 2, :, :] = _wrap(opt_ref[0, :, :] - oi_col)


def kernel(pos_pt, orient_pt, pos_pl, orient_pl, valid_pl):
    B, PL, PT, _ = pos_pt.shape
    R = 512

    ptxy = jnp.concatenate([pos_pt[..., 0], pos_pt[..., 1]], axis=-1) * 0.1
    prm = jnp.stack(
        [pos_pl[..., 0] * 0.1, pos_pl[..., 1] * 0.1, orient_pl,
         valid_pl.astype(jnp.float32)], axis=1)

    prm_spec = pl.BlockSpec((1, 4, PL), lambda b, r: (b, 0, 0))
    ptxy_spec = pl.BlockSpec((1, R, 2 * PT), lambda b, r: (b, r, 0))
    opt_spec = pl.BlockSpec((1, R, PT), lambda b, r: (b, r, 0))

    r_pl2pl, r_pt2pl, mask = pl.pallas_call(
        _geom_kernel,
        grid=(B, PL // R),
        in_specs=[prm_spec, ptxy_spec, opt_spec],
        out_specs=[
            pl.BlockSpec((1, 3, R, PL), lambda b, r: (b, 0, r, 0)),
            pl.BlockSpec((1, 3, R, PT), lambda b, r: (b, 0, r, 0)),
            pl.BlockSpec((1, R, PL), lambda b, r: (b, r, 0)),
        ],
        out_shape=(
            jax.ShapeDtypeStruct((B, 3, PL, PL), jnp.float32),
            jax.ShapeDtypeStruct((B, 3, PL, PT), jnp.float32),
            jax.ShapeDtypeStruct((B, PL, PL), jnp.bool_),
        ),
        compiler_params=pltpu.CompilerParams(
            dimension_semantics=("parallel", "parallel"),
        ),
    )(prm, ptxy, orient_pt)

    return (r_pl2pl, r_pt2pl, mask)
